# baseline (speedup 1.0000x reference)
import jax
import jax.numpy as jnp
from jax import lax
from jax.experimental import pallas as pl
from jax.experimental.pallas import tpu as pltpu
from jax.experimental.pallas import tpu_sc as plsc

SEQ_LEN = 200
DIM = 64
NC = 2
NS = 16
NW = NC * NS
BCH = 128
RW = 128
RP = 129
NST = 4
NIB = 2
NRB = 2
GA = 1
NSB = 2
NPB = 2
RING = 2


def _body(idx_hbm, par_hbm, tab_hbm, posb_hbm, out_hbm,
          idx_bufs, par_bufs, row_bufs, slab_bufs, posb_bufs,
          isems, asems, gsems, ssems, psems):
    wid = lax.axis_index("s") * NC + lax.axis_index("c")
    lanes = lax.iota(jnp.int32, 16)

    def idx_fire(l, s):
        pltpu.async_copy(idx_hbm.at[l, pl.ds(wid * BCH, BCH)],
                         idx_bufs[s], isems[s])

    def idx_wait(l, s):
        pltpu.make_async_copy(idx_hbm.at[l, pl.ds(wid * BCH, BCH)],
                              idx_bufs[s], isems[s]).wait()

    def par_fire(l, s):
        pltpu.async_copy(par_hbm.at[l, pl.ds(wid * BCH, BCH)],
                         par_bufs[s], asems[s])

    def par_wait(l, s):
        pltpu.make_async_copy(par_hbm.at[l, pl.ds(wid * BCH, BCH)],
                              par_bufs[s], asems[s]).wait()

    def gather_fire(si, sr):
        n = BCH // NST
        for t in range(NST):
            pltpu.async_copy(
                tab_hbm.at[idx_bufs[si].at[pl.ds(t * n, n)]],
                row_bufs[sr].at[pl.ds(t * n, n), pl.ds(0, RW)], gsems[sr])

    def gather_wait(si, sr):
        pltpu.make_async_copy(tab_hbm.at[idx_bufs[si]],
                              row_bufs[sr].at[pl.ds(0, BCH), pl.ds(0, RW)],
                              gsems[sr]).wait()

    def posb_fire(l, s):
        pltpu.async_copy(posb_hbm.at[l], posb_bufs[s], psems[s])

    def posb_wait(l, s):
        pltpu.make_async_copy(posb_hbm.at[l], posb_bufs[s], psems[s]).wait()

    def store_fire(l, s):
        pltpu.async_copy(slab_bufs[s],
                         out_hbm.at[l, :, pl.ds(wid * BCH, BCH)], ssems[s])

    def store_wait(l, s):
        pltpu.make_async_copy(slab_bufs[s],
                              out_hbm.at[l, :, pl.ds(wid * BCH, BCH)],
                              ssems[s]).wait()

    for m in range(NIB):
        idx_fire(m, m)
    for m in range(GA):
        idx_wait(m, m)
        gather_fire(m, m % NRB)
    for m in range(NPB):
        posb_fire(m, m)
        par_fire(m, m)

    rowvs = tuple(g * 16 + lanes for g in range(BCH // 16))

    def one_round(r, carry):
        for j in range(RING):
            l = r * RING + j
            si, sr, ss, sp = j % NIB, j % NRB, j % NSB, j % NPB

            gather_wait(si, sr)
            posb_wait(l, sp)
            par_wait(l, sp)

            @pl.when(l + NIB < SEQ_LEN)
            def _():
                idx_fire(l + NIB, si)

            @pl.when(l + GA < SEQ_LEN)
            def _():
                smi = (j + GA) % NIB
                idx_wait(l + GA, smi)
                gather_fire(smi, (j + GA) % NRB)

            @pl.when(l >= NSB)
            def _():
                store_wait(l - NSB, ss)

            rows = row_bufs[sr]
            slab = slab_bufs[ss]
            posb = posb_bufs[sp]
            parb = par_bufs[sp]

            parv = tuple(parb[pl.ds(g * 16, 16)] for g in range(BCH // 16))

            for d in range(DIM):
                pv = posb[pl.ds(d * 16, 16)]
                for g in range(BCH // 16):
                    x = plsc.load_gather(rows, [rowvs[g], parv[g] + d])
                    slab[d, pl.ds(g * 16, 16)] = x + pv

            store_fire(l, ss)

            @pl.when(l + NPB < SEQ_LEN)
            def _():
                posb_fire(l + NPB, sp)
                par_fire(l + NPB, sp)
        return carry

    lax.fori_loop(0, SEQ_LEN // RING, one_round, 0)

    for m in range(NSB):
        store_wait(SEQ_LEN - NSB + m, (SEQ_LEN - NSB + m) % NSB)


def kernel(inputs, token_table, pos_table):
    batch, seq_len = inputs.shape
    vocab = token_table.shape[0]
    assert seq_len == SEQ_LEN and batch == NW * BCH and vocab % 2 == 0
    ids = inputs.astype(jnp.int32)
    ids2 = (ids >> 1).T
    par = ((ids & 1) << 6).T
    tab2 = token_table.reshape(vocab // 2, 2 * DIM)
    posb = jnp.broadcast_to(pos_table[:, :, None],
                            (SEQ_LEN, DIM, 16)).reshape(SEQ_LEN, DIM * 16)

    mesh = plsc.VectorSubcoreMesh(
        core_axis_name="c", subcore_axis_name="s",
        num_cores=NC, num_subcores=NS)

    run = pl.kernel(
        _body,
        out_type=jax.ShapeDtypeStruct((SEQ_LEN, DIM, batch), jnp.float32),
        mesh=mesh,
        scratch_types=[
            [pltpu.VMEM((BCH,), jnp.int32) for _ in range(NIB)],
            [pltpu.VMEM((BCH,), jnp.int32) for _ in range(NPB)],
            [pltpu.VMEM((BCH, RP), jnp.float32) for _ in range(NRB)],
            [pltpu.VMEM((DIM, BCH), jnp.float32) for _ in range(NSB)],
            [pltpu.VMEM((DIM * 16,), jnp.float32) for _ in range(NPB)],
            [pltpu.SemaphoreType.DMA for _ in range(NIB)],
            [pltpu.SemaphoreType.DMA for _ in range(NPB)],
            [pltpu.SemaphoreType.DMA for _ in range(NRB)],
            [pltpu.SemaphoreType.DMA for _ in range(NSB)],
            [pltpu.SemaphoreType.DMA for _ in range(NPB)],
        ],
        compiler_params=pltpu.CompilerParams(
            use_tc_tiling_on_sc=True, needs_layout_passes=False),
    )
    out_phys = run(ids2, par, tab2, posb)
    return out_phys.transpose(2, 0, 1)


# speedup vs baseline: 1.1132x; 1.1132x over previous
import jax
import jax.numpy as jnp
from jax import lax
from jax.experimental import pallas as pl
from jax.experimental.pallas import tpu as pltpu
from jax.experimental.pallas import tpu_sc as plsc

SEQ_LEN = 200
DIM = 64
NC = 2
NS = 16
NW = NC * NS
BCH = 128
RW = 128
NG = BCH // 16
NIB = 8
NRB = 4
GA = 3
NSB = 2
NPB = 4
RING = 8


def _body(idx_hbm, tab_hbm, posb_hbm, out_hbm,
          idx_bufs, shift_bufs, row_bufs, slab_bufs, posb_bufs,
          isems, gsems, ssems, psems):
    wid = lax.axis_index("s") * NC + lax.axis_index("c")
    lanes = lax.iota(jnp.int32, 16)

    def idx_fire(l, s):
        pltpu.async_copy(idx_hbm.at[l, pl.ds(wid * BCH, BCH)],
                         idx_bufs[s], isems[s])

    def idx_wait(l, s):
        pltpu.make_async_copy(idx_hbm.at[l, pl.ds(wid * BCH, BCH)],
                              idx_bufs[s], isems[s]).wait()

    def shift_and_gather_fire(si, sr):
        for g in range(NG):
            v = idx_bufs[si][pl.ds(g * 16, 16)]
            shift_bufs[sr][pl.ds(g * 16, 16)] = lax.shift_right_logical(v, 1)
        pltpu.async_copy(tab_hbm.at[shift_bufs[sr]], row_bufs[sr], gsems[sr])

    def gather_wait(sr):
        pltpu.make_async_copy(tab_hbm.at[shift_bufs[sr]], row_bufs[sr],
                              gsems[sr]).wait()

    def posb_fire(l, s):
        pltpu.async_copy(posb_hbm.at[l], posb_bufs[s], psems[s])

    def posb_wait(l, s):
        pltpu.make_async_copy(posb_hbm.at[l], posb_bufs[s], psems[s]).wait()

    def store_fire(l, s):
        pltpu.async_copy(slab_bufs[s],
                         out_hbm.at[l, :, pl.ds(wid * BCH, BCH)], ssems[s])

    def store_wait(l, s):
        pltpu.make_async_copy(slab_bufs[s],
                              out_hbm.at[l, :, pl.ds(wid * BCH, BCH)],
                              ssems[s]).wait()

    for m in range(NIB):
        idx_fire(m, m)
    for m in range(GA):
        idx_wait(m, m)
        shift_and_gather_fire(m, m % NRB)
    for m in range(NPB):
        posb_fire(m, m)

    rowvs = tuple(g * 16 + lanes for g in range(NG))

    def one_round(r, carry):
        for j in range(RING):
            l = r * RING + j
            si, sr, ss, sp = j % NIB, j % NRB, j % NSB, j % NPB

            gather_wait(sr)
            posb_wait(l, sp)

            parv = tuple(
                lax.shift_left(
                    lax.bitwise_and(idx_bufs[si][pl.ds(g * 16, 16)], 1), 6)
                for g in range(NG))

            @pl.when(l + NIB < SEQ_LEN)
            def _():
                idx_fire(l + NIB, si)

            @pl.when(l + GA < SEQ_LEN)
            def _():
                smi = (j + GA) % NIB
                idx_wait(l + GA, smi)
                shift_and_gather_fire(smi, (j + GA) % NRB)

            @pl.when(l >= NSB)
            def _():
                store_wait(l - NSB, ss)

            rows = row_bufs[sr]
            slab = slab_bufs[ss]
            posb = posb_bufs[sp]

            def one_d(d, c):
                pv = posb[pl.ds(d * 16, 16)]
                for g in range(NG):
                    x = plsc.load_gather(rows, [rowvs[g], parv[g] + d])
                    slab[d, pl.ds(g * 16, 16)] = x + pv
                return c

            lax.fori_loop(0, DIM, one_d, 0, unroll=8)

            store_fire(l, ss)

            @pl.when(l + NPB < SEQ_LEN)
            def _():
                posb_fire(l + NPB, sp)
        return carry

    lax.fori_loop(0, SEQ_LEN // RING, one_round, 0)

    for m in range(NSB):
        store_wait(SEQ_LEN - NSB + m, (SEQ_LEN - NSB + m) % NSB)


def kernel(inputs, token_table, pos_table):
    batch, seq_len = inputs.shape
    vocab = token_table.shape[0]
    assert seq_len == SEQ_LEN and batch == NW * BCH and vocab % 2 == 0
    idx_t = inputs.astype(jnp.int32).T
    tab2 = token_table.reshape(vocab // 2, 2 * DIM)
    posb = jnp.broadcast_to(pos_table[:, :, None],
                            (SEQ_LEN, DIM, 16)).reshape(SEQ_LEN, DIM * 16)

    mesh = plsc.VectorSubcoreMesh(
        core_axis_name="c", subcore_axis_name="s",
        num_cores=NC, num_subcores=NS)

    run = pl.kernel(
        _body,
        out_type=jax.ShapeDtypeStruct((SEQ_LEN, DIM, batch), jnp.float32),
        mesh=mesh,
        scratch_types=[
            [pltpu.VMEM((BCH,), jnp.int32) for _ in range(NIB)],
            [pltpu.VMEM((BCH,), jnp.int32) for _ in range(NRB)],
            [pltpu.VMEM((BCH, RW), jnp.float32) for _ in range(NRB)],
            [pltpu.VMEM((DIM, BCH), jnp.float32) for _ in range(NSB)],
            [pltpu.VMEM((DIM * 16,), jnp.float32) for _ in range(NPB)],
            [pltpu.SemaphoreType.DMA for _ in range(NIB)],
            [pltpu.SemaphoreType.DMA for _ in range(NRB)],
            [pltpu.SemaphoreType.DMA for _ in range(NSB)],
            [pltpu.SemaphoreType.DMA for _ in range(NPB)],
        ],
        compiler_params=pltpu.CompilerParams(
            use_tc_tiling_on_sc=True, needs_layout_passes=False),
    )
    out_phys = run(idx_t, tab2, posb)
    return out_phys.transpose(2, 0, 1)


# speedup vs baseline: 1.7173x; 1.5426x over previous
import jax
import jax.numpy as jnp
from jax import lax
from jax.experimental import pallas as pl
from jax.experimental.pallas import tpu as pltpu
from jax.experimental.pallas import tpu_sc as plsc

SEQ_LEN = 200
DIM = 64
NC = 2
NS = 16
NW = NC * NS
BCH = 128
RW = 128
NG = BCH // 16
NIB = 8
NRB = 4
GA = 3
NSB = 2
NPB = 4
RING = 8


def _body(idx_hbm, tab_hbm, posb_hbm, out_hbm,
          idx_bufs, shift_bufs, row_bufs, slab_bufs, posb_bufs,
          isems, gsems, ssems, psems):
    wid = lax.axis_index("s") * NC + lax.axis_index("c")
    lanes = lax.iota(jnp.int32, 16)

    def idx_fire(l, s):
        pltpu.async_copy(idx_hbm.at[l, pl.ds(wid * BCH, BCH)],
                         idx_bufs[s], isems[s])

    def idx_wait(l, s):
        pltpu.make_async_copy(idx_hbm.at[l, pl.ds(wid * BCH, BCH)],
                              idx_bufs[s], isems[s]).wait()

    def shift_and_gather_fire(si, sr):
        for g in range(NG):
            v = idx_bufs[si][pl.ds(g * 16, 16)]
            shift_bufs[sr][pl.ds(g * 16, 16)] = lax.shift_right_logical(v, 1)
        pltpu.async_copy(tab_hbm.at[shift_bufs[sr]], row_bufs[sr], gsems[sr])

    def gather_wait(sr):
        pltpu.make_async_copy(tab_hbm.at[shift_bufs[sr]], row_bufs[sr],
                              gsems[sr]).wait()

    def posb_fire(l, s):
        pltpu.async_copy(posb_hbm.at[l], posb_bufs[s], psems[s])

    def posb_wait(l, s):
        pltpu.make_async_copy(posb_hbm.at[l], posb_bufs[s], psems[s]).wait()

    def store_fire(l, s):
        pltpu.async_copy(slab_bufs[s],
                         out_hbm.at[l, :, pl.ds(wid * BCH, BCH)], ssems[s])

    def store_wait(l, s):
        pltpu.make_async_copy(slab_bufs[s],
                              out_hbm.at[l, :, pl.ds(wid * BCH, BCH)],
                              ssems[s]).wait()

    for m in range(NIB):
        idx_fire(m, m)
    for m in range(GA):
        idx_wait(m, m)
        shift_and_gather_fire(m, m % NRB)
    for m in range(NPB):
        posb_fire(m, m)

    rowvs = tuple(g * 16 + lanes for g in range(NG))

    def one_round(r, carry):
        for j in range(RING):
            l = r * RING + j
            si, sr, ss, sp = j % NIB, j % NRB, j % NSB, j % NPB

            gather_wait(sr)
            posb_wait(l, sp)

            parv = tuple(
                lax.shift_left(
                    lax.bitwise_and(idx_bufs[si][pl.ds(g * 16, 16)], 1), 6)
                for g in range(NG))

            @pl.when(l + NIB < SEQ_LEN)
            def _():
                idx_fire(l + NIB, si)

            @pl.when(l + GA < SEQ_LEN)
            def _():
                smi = (j + GA) % NIB
                idx_wait(l + GA, smi)
                shift_and_gather_fire(smi, (j + GA) % NRB)

            @pl.when(l >= NSB)
            def _():
                store_wait(l - NSB, ss)

            rows = row_bufs[sr]
            slab = slab_bufs[ss]
            posb = posb_bufs[sp]

            @plsc.parallel_loop(0, DIM, unroll=4)
            def _(d):
                pv = posb[pl.ds(d * 16, 16)]
                for g in range(NG):
                    x = plsc.load_gather(rows, [rowvs[g], parv[g] + d])
                    slab[d, pl.ds(g * 16, 16)] = x + pv

            store_fire(l, ss)

            @pl.when(l + NPB < SEQ_LEN)
            def _():
                posb_fire(l + NPB, sp)
        return carry

    lax.fori_loop(0, SEQ_LEN // RING, one_round, 0)

    for m in range(NSB):
        store_wait(SEQ_LEN - NSB + m, (SEQ_LEN - NSB + m) % NSB)


def kernel(inputs, token_table, pos_table):
    batch, seq_len = inputs.shape
    vocab = token_table.shape[0]
    assert seq_len == SEQ_LEN and batch == NW * BCH and vocab % 2 == 0
    idx_t = inputs.astype(jnp.int32).T
    tab2 = token_table.reshape(vocab // 2, 2 * DIM)
    posb = jnp.broadcast_to(pos_table[:, :, None],
                            (SEQ_LEN, DIM, 16)).reshape(SEQ_LEN, DIM * 16)

    mesh = plsc.VectorSubcoreMesh(
        core_axis_name="c", subcore_axis_name="s",
        num_cores=NC, num_subcores=NS)

    run = pl.kernel(
        _body,
        out_type=jax.ShapeDtypeStruct((SEQ_LEN, DIM, batch), jnp.float32),
        mesh=mesh,
        scratch_types=[
            [pltpu.VMEM((BCH,), jnp.int32) for _ in range(NIB)],
            [pltpu.VMEM((BCH,), jnp.int32) for _ in range(NRB)],
            [pltpu.VMEM((BCH, RW), jnp.float32) for _ in range(NRB)],
            [pltpu.VMEM((DIM, BCH), jnp.float32) for _ in range(NSB)],
            [pltpu.VMEM((DIM * 16,), jnp.float32) for _ in range(NPB)],
            [pltpu.SemaphoreType.DMA for _ in range(NIB)],
            [pltpu.SemaphoreType.DMA for _ in range(NRB)],
            [pltpu.SemaphoreType.DMA for _ in range(NSB)],
            [pltpu.SemaphoreType.DMA for _ in range(NPB)],
        ],
        compiler_params=pltpu.CompilerParams(
            use_tc_tiling_on_sc=True, needs_layout_passes=False),
    )
    out_phys = run(idx_t, tab2, posb)
    return out_phys.transpose(2, 0, 1)


# speedup vs baseline: 1.7176x; 1.0002x over previous
import jax
import jax.numpy as jnp
from jax import lax
from jax.experimental import pallas as pl
from jax.experimental.pallas import tpu as pltpu
from jax.experimental.pallas import tpu_sc as plsc

SEQ_LEN = 200
DIM = 64
NC = 2
NS = 16
NW = NC * NS
BCH = 128
RW = 128
NG = BCH // 16
NIB = 8
NRB = 4
GA = 3
NSB = 2
NPB = 4
RING = 8


def _body(idx_hbm, tab_hbm, posb_hbm, out_hbm,
          idx_bufs, shift_bufs, row_bufs, slab_bufs, posb_bufs,
          isems, gsems, ssems, psems):
    wid = lax.axis_index("s") * NC + lax.axis_index("c")
    lanes = lax.iota(jnp.int32, 16)

    def idx_fire(l, s):
        pltpu.async_copy(idx_hbm.at[l, pl.ds(wid * BCH, BCH)],
                         idx_bufs[s], isems[s])

    def idx_wait(l, s):
        pltpu.make_async_copy(idx_hbm.at[l, pl.ds(wid * BCH, BCH)],
                              idx_bufs[s], isems[s]).wait()

    def shift_and_gather_fire(si, sr):
        for g in range(NG):
            v = idx_bufs[si][pl.ds(g * 16, 16)]
            shift_bufs[sr][pl.ds(g * 16, 16)] = lax.shift_right_logical(v, 1)
        pltpu.async_copy(tab_hbm.at[shift_bufs[sr]], row_bufs[sr], gsems[sr])

    def gather_wait(sr):
        pltpu.make_async_copy(tab_hbm.at[shift_bufs[sr]], row_bufs[sr],
                              gsems[sr]).wait()

    def posb_fire(l, s):
        pltpu.async_copy(posb_hbm.at[l], posb_bufs[s], psems[s])

    def posb_wait(l, s):
        pltpu.make_async_copy(posb_hbm.at[l], posb_bufs[s], psems[s]).wait()

    def store_fire(l, s):
        pltpu.async_copy(slab_bufs[s],
                         out_hbm.at[l, :, pl.ds(wid * BCH, BCH)], ssems[s])

    def store_wait(l, s):
        pltpu.make_async_copy(slab_bufs[s],
                              out_hbm.at[l, :, pl.ds(wid * BCH, BCH)],
                              ssems[s]).wait()

    for m in range(NIB):
        idx_fire(m, m)
    for m in range(GA):
        idx_wait(m, m)
        shift_and_gather_fire(m, m % NRB)
    for m in range(NPB):
        posb_fire(m, m)

    rowvs = tuple(g * 16 + lanes for g in range(NG))

    def one_round(r, carry):
        for j in range(RING):
            l = r * RING + j
            si, sr, ss, sp = j % NIB, j % NRB, j % NSB, j % NPB

            gather_wait(sr)
            posb_wait(l, sp)

            parv = tuple(
                lax.shift_left(
                    lax.bitwise_and(idx_bufs[si][pl.ds(g * 16, 16)], 1), 6)
                for g in range(NG))

            @pl.when(l + NIB < SEQ_LEN)
            def _():
                idx_fire(l + NIB, si)

            @pl.when(l + GA < SEQ_LEN)
            def _():
                smi = (j + GA) % NIB
                idx_wait(l + GA, smi)
                shift_and_gather_fire(smi, (j + GA) % NRB)

            @pl.when(l >= NSB)
            def _():
                store_wait(l - NSB, ss)

            rows = row_bufs[sr]
            slab = slab_bufs[ss]
            posb = posb_bufs[sp]

            @plsc.parallel_loop(0, DIM, unroll=8)
            def _(d):
                pv = posb[pl.ds(d * 16, 16)]
                for g in range(NG):
                    x = plsc.load_gather(rows, [rowvs[g], parv[g] + d])
                    slab[d, pl.ds(g * 16, 16)] = x + pv

            store_fire(l, ss)

            @pl.when(l + NPB < SEQ_LEN)
            def _():
                posb_fire(l + NPB, sp)
        return carry

    lax.fori_loop(0, SEQ_LEN // RING, one_round, 0)

    for m in range(NSB):
        store_wait(SEQ_LEN - NSB + m, (SEQ_LEN - NSB + m) % NSB)


def kernel(inputs, token_table, pos_table):
    batch, seq_len = inputs.shape
    vocab = token_table.shape[0]
    assert seq_len == SEQ_LEN and batch == NW * BCH and vocab % 2 == 0
    idx_t = inputs.astype(jnp.int32).T
    tab2 = token_table.reshape(vocab // 2, 2 * DIM)
    posb = jnp.broadcast_to(pos_table[:, :, None],
                            (SEQ_LEN, DIM, 16)).reshape(SEQ_LEN, DIM * 16)

    mesh = plsc.VectorSubcoreMesh(
        core_axis_name="c", subcore_axis_name="s",
        num_cores=NC, num_subcores=NS)

    run = pl.kernel(
        _body,
        out_type=jax.ShapeDtypeStruct((SEQ_LEN, DIM, batch), jnp.float32),
        mesh=mesh,
        scratch_types=[
            [pltpu.VMEM((BCH,), jnp.int32) for _ in range(NIB)],
            [pltpu.VMEM((BCH,), jnp.int32) for _ in range(NRB)],
            [pltpu.VMEM((BCH, RW), jnp.float32) for _ in range(NRB)],
            [pltpu.VMEM((DIM, BCH), jnp.float32) for _ in range(NSB)],
            [pltpu.VMEM((DIM * 16,), jnp.float32) for _ in range(NPB)],
            [pltpu.SemaphoreType.DMA for _ in range(NIB)],
            [pltpu.SemaphoreType.DMA for _ in range(NRB)],
            [pltpu.SemaphoreType.DMA for _ in range(NSB)],
            [pltpu.SemaphoreType.DMA for _ in range(NPB)],
        ],
        compiler_params=pltpu.CompilerParams(
            use_tc_tiling_on_sc=True, needs_layout_passes=False),
    )
    out_phys = run(idx_t, tab2, posb)
    return out_phys.transpose(2, 0, 1)
